# Initial kernel scaffold; baseline (speedup 1.0000x reference)
import jax
import jax.numpy as jnp
from jax.experimental import pallas as pl


def kernel(x_soft, x_feat, y):
    raise NotImplementedError("write your pallas kernel here")


# speedup vs baseline: 1.7838x; 1.7838x over previous
import jax
import jax.numpy as jnp
from jax.experimental import pallas as pl
from jax.experimental.pallas import tpu as pltpu

_LAMDA = 0.003
_K = 20
_N = 1024
_RB = 256
_KC = 1024
_DPAD = 2048
_NRB = _N // _RB
_NKB = _DPAD // _KC


def _loss_kernel(y_col_ref, y_row_ref, xs_ref, xb_ref, xT_ref, out_ref,
                 g_ref, sq_all_ref, sq_blk_ref):
    rb = pl.program_id(0)
    kb = pl.program_id(1)

    @pl.when((rb == 0) & (kb == 0))
    def _init_out():
        out_ref[...] = jnp.zeros((1, 1), jnp.float32)

    xb = xb_ref[...]
    xT = xT_ref[...]
    part = jnp.dot(xb, xT, preferred_element_type=jnp.float32)
    pb = jnp.sum(xb * xb, axis=1, keepdims=True)

    @pl.when(kb == 0)
    def _init_acc():
        g_ref[...] = part
        sq_blk_ref[...] = pb

    @pl.when(kb != 0)
    def _acc():
        g_ref[...] += part
        sq_blk_ref[...] += pb

    @pl.when(rb == 0)
    def _sq_all():
        pa = jnp.sum(xT * xT, axis=0, keepdims=True)

        @pl.when(kb == 0)
        def _():
            sq_all_ref[...] = pa

        @pl.when(kb != 0)
        def _():
            sq_all_ref[...] += pa

    @pl.when(kb == _NKB - 1)
    def _finish_block():
        g = g_ref[...]
        sq_b = sq_blk_ref[...]
        sq_a = sq_all_ref[...]
        d = sq_b + sq_a - 2.0 * g

        y_b = y_col_ref[...]
        y_a = y_row_ref[...]
        col = jax.lax.broadcasted_iota(jnp.int32, (_RB, _N), 1)
        row_g = jax.lax.broadcasted_iota(jnp.int32, (_RB, _N), 0) + rb * _RB
        valid = (y_b == y_a) & (col != row_g)
        dm0 = jnp.where(valid, d, jnp.inf)

        def body(_, carry):
            dm, acc = carry
            m = jnp.min(dm, axis=1, keepdims=True)
            acc = acc + jnp.where(jnp.isfinite(m), m, 0.0)
            first = jnp.min(jnp.where(dm == m, col, _N), axis=1,
                            keepdims=True)
            dm = jnp.where(col == first, jnp.inf, dm)
            return dm, acc

        _, acc = jax.lax.fori_loop(
            0, _K, body, (dm0, jnp.zeros((_RB, 1), jnp.float32)))
        lp = jnp.sum(acc, keepdims=True)

        xs = xs_ref[...]
        mx = jnp.max(xs, axis=1, keepdims=True)
        lse = mx + jnp.log(jnp.sum(jnp.exp(xs - mx), axis=1, keepdims=True))
        lane = jax.lax.broadcasted_iota(jnp.int32, xs.shape, 1)
        lab = jnp.sum(jnp.where(lane == y_b, xs, 0.0), axis=1, keepdims=True)
        ce = jnp.sum(lse - lab, keepdims=True)

        out_ref[...] += (_LAMDA / 2.0) * lp + ce / _N


def kernel(x_soft, x_feat, y):
    n, d = x_feat.shape
    xf = jnp.pad(x_feat, ((0, 0), (0, _DPAD - d)))
    xT = xf.T
    xs = jnp.pad(x_soft, ((0, 0), (0, 128 - x_soft.shape[1])),
                 constant_values=-1e30)
    y = y.astype(jnp.int32)
    y_col = y[:, None]
    y_row = y[None, :]

    out = pl.pallas_call(
        _loss_kernel,
        grid=(_NRB, _NKB),
        in_specs=[
            pl.BlockSpec((_RB, 1), lambda rb, kb: (rb, 0)),
            pl.BlockSpec((1, _N), lambda rb, kb: (0, 0)),
            pl.BlockSpec((_RB, 128), lambda rb, kb: (rb, 0)),
            pl.BlockSpec((_RB, _KC), lambda rb, kb: (rb, kb)),
            pl.BlockSpec((_KC, _N), lambda rb, kb: (kb, 0)),
        ],
        out_specs=pl.BlockSpec((1, 1), lambda rb, kb: (0, 0)),
        out_shape=jax.ShapeDtypeStruct((1, 1), jnp.float32),
        scratch_shapes=[
            pltpu.VMEM((_RB, _N), jnp.float32),
            pltpu.VMEM((1, _N), jnp.float32),
            pltpu.VMEM((_RB, 1), jnp.float32),
        ],
    )(y_col, y_row, xs, xf, xT)
    return out[0, 0]


# speedup vs baseline: 4.2435x; 2.3789x over previous
import jax
import jax.numpy as jnp
from jax.experimental import pallas as pl

_LAMDA = 0.003
_K = 20
_N = 1024
_CLS = 128
_DPAD = 2048


def _loss_kernel(y_col_ref, y_row_ref, xs_ref, x_ref, out_ref):
    x = x_ref[...]
    y_col = y_col_ref[...]
    y_row = y_row_ref[...]

    cls_iota = jax.lax.broadcasted_iota(jnp.int32, (_CLS, _N), 0)
    oh = (cls_iota == y_row).astype(jnp.float32)
    s_cls = jnp.dot(oh, x, preferred_element_type=jnp.float32)
    p = jax.lax.dot_general(x, s_cls, (((1,), (1,)), ((), ())),
                            preferred_element_type=jnp.float32)

    sq = jnp.sum(x * x, axis=1, keepdims=True)
    same = y_col == y_row
    mf = same.astype(jnp.float32)
    cnt = jnp.sum(mf, axis=1, keepdims=True)
    ssq_cls = jnp.dot(mf, sq, preferred_element_type=jnp.float32)

    lane_oh = jax.lax.broadcasted_iota(jnp.int32, (_N, _CLS), 1) == y_col
    t = jnp.sum(jnp.where(lane_oh, p, 0.0), axis=1, keepdims=True)

    rowsum = cnt * sq + ssq_cls - 2.0 * t
    lp_base = jnp.sum(rowsum, keepdims=True)

    excess0 = jnp.maximum(cnt - 1.0 - _K, 0.0)
    col = jax.lax.broadcasted_iota(jnp.int32, (_N, _N), 1)
    row = jax.lax.broadcasted_iota(jnp.int32, (_N, _N), 0)

    def _heavy():
        g = jax.lax.dot_general(x, x, (((1,), (1,)), ((), ())),
                                preferred_element_type=jnp.float32)
        diag_row = jnp.sum(jnp.where(col == row, g, 0.0), axis=0,
                           keepdims=True)
        d = sq + diag_row - 2.0 * g
        valid = same & (col != row)
        dmn0 = jnp.where(valid, d, -jnp.inf)

        def cond(carry):
            return jnp.max(carry[1]) > 0.0

        def body(carry):
            dmn, ex, corr = carry
            m = jnp.max(dmn, axis=1, keepdims=True)
            corr = corr + jnp.sum(jnp.where(ex > 0.0, m, 0.0), keepdims=True)
            first = jnp.min(jnp.where(dmn == m, col, _N), axis=1,
                            keepdims=True)
            dmn = jnp.where((col == first) & (ex > 0.0), -jnp.inf, dmn)
            return dmn, jnp.maximum(ex - 1.0, 0.0), corr

        _, _, corr = jax.lax.while_loop(
            cond, body, (dmn0, excess0, jnp.zeros((1, 1), jnp.float32)))
        return corr

    corr = jax.lax.cond(jnp.max(excess0) > 0.0, _heavy,
                        lambda: jnp.zeros((1, 1), jnp.float32))

    xs = xs_ref[...]
    mx = jnp.max(xs, axis=1, keepdims=True)
    lse = mx + jnp.log(jnp.sum(jnp.exp(xs - mx), axis=1, keepdims=True))
    lab = jnp.sum(jnp.where(lane_oh, xs, 0.0), axis=1, keepdims=True)
    ce = jnp.sum(lse - lab, keepdims=True)

    out_ref[...] = (_LAMDA / 2.0) * (lp_base - corr) + ce / _N


def kernel(x_soft, x_feat, y):
    n, d = x_feat.shape
    xf = jnp.pad(x_feat, ((0, 0), (0, _DPAD - d)))
    xs = jnp.pad(x_soft, ((0, 0), (0, _CLS - x_soft.shape[1])),
                 constant_values=-1e30)
    y = y.astype(jnp.int32)

    out = pl.pallas_call(
        _loss_kernel,
        in_specs=[
            pl.BlockSpec((_N, 1), lambda: (0, 0)),
            pl.BlockSpec((1, _N), lambda: (0, 0)),
            pl.BlockSpec((_N, _CLS), lambda: (0, 0)),
            pl.BlockSpec((_N, _DPAD), lambda: (0, 0)),
        ],
        out_specs=pl.BlockSpec((1, 1), lambda: (0, 0)),
        out_shape=jax.ShapeDtypeStruct((1, 1), jnp.float32),
    )(y[:, None], y[None, :], xs, xf)
    return out[0, 0]


# speedup vs baseline: 9.0669x; 2.1367x over previous
import jax
import jax.numpy as jnp
from jax.experimental import pallas as pl

_LAMDA = 0.003
_K = 20
_N = 1024
_CLS = 128


def _loss_kernel(y_col_ref, y_row_ref, xs_ref, x_ref, out_ref):
    xb = x_ref[...]
    y_col = y_col_ref[...]
    y_row = y_row_ref[...]

    cls_iota = jax.lax.broadcasted_iota(jnp.int32, (_CLS, _N), 0)
    oh = cls_iota == y_row
    ohf = oh.astype(jnp.float32)
    cnt_c = jnp.sum(ohf, axis=1, keepdims=True)

    s_cls = jnp.dot(oh.astype(jnp.bfloat16), xb,
                    preferred_element_type=jnp.float32)

    xf = xb.astype(jnp.float32)
    sq = jnp.sum(xf * xf, axis=1, keepdims=True)
    ssq_c = jnp.dot(ohf, sq, preferred_element_type=jnp.float32)

    term1 = jnp.sum(cnt_c * ssq_c, keepdims=True)
    term2 = jnp.sum(s_cls * s_cls, keepdims=True)
    lp_base = 2.0 * (term1 - term2)

    def _heavy():
        col = jax.lax.broadcasted_iota(jnp.int32, (_N, _N), 1)
        row = jax.lax.broadcasted_iota(jnp.int32, (_N, _N), 0)
        same = y_col == y_row
        cnt_i = jnp.sum(same.astype(jnp.float32), axis=1, keepdims=True)
        excess0 = jnp.maximum(cnt_i - 1.0 - _K, 0.0)

        g = jax.lax.dot_general(xb, xb, (((1,), (1,)), ((), ())),
                                preferred_element_type=jnp.float32)
        diag_row = jnp.sum(jnp.where(col == row, g, 0.0), axis=0,
                           keepdims=True)
        d = sq + diag_row - 2.0 * g
        valid = same & (col != row)
        dmn0 = jnp.where(valid, d, -jnp.inf)

        def cond(carry):
            return jnp.max(carry[1]) > 0.0

        def body(carry):
            dmn, ex, corr = carry
            m = jnp.max(dmn, axis=1, keepdims=True)
            corr = corr + jnp.sum(jnp.where(ex > 0.0, m, 0.0), keepdims=True)
            first = jnp.min(jnp.where(dmn == m, col, _N), axis=1,
                            keepdims=True)
            dmn = jnp.where((col == first) & (ex > 0.0), -jnp.inf, dmn)
            return dmn, jnp.maximum(ex - 1.0, 0.0), corr

        _, _, corr = jax.lax.while_loop(
            cond, body, (dmn0, excess0, jnp.zeros((1, 1), jnp.float32)))
        return corr

    corr = jax.lax.cond(jnp.max(cnt_c) > _K + 1.0, _heavy,
                        lambda: jnp.zeros((1, 1), jnp.float32))

    xs = xs_ref[...]
    mx = jnp.max(xs, axis=1, keepdims=True)
    lse = mx + jnp.log(jnp.sum(jnp.exp(xs - mx), axis=1, keepdims=True))
    lane = jax.lax.broadcasted_iota(jnp.int32, xs.shape, 1)
    lab = jnp.sum(jnp.where(lane == y_col, xs, 0.0), axis=1, keepdims=True)
    ce = jnp.sum(lse - lab, keepdims=True)

    out_ref[...] = (_LAMDA / 2.0) * (lp_base - corr) + ce / _N


def kernel(x_soft, x_feat, y):
    n, d = x_feat.shape
    xb = x_feat.astype(jnp.bfloat16)
    y = y.astype(jnp.int32)

    out = pl.pallas_call(
        _loss_kernel,
        in_specs=[
            pl.BlockSpec((_N, 1), lambda: (0, 0)),
            pl.BlockSpec((1, _N), lambda: (0, 0)),
            pl.BlockSpec(x_soft.shape, lambda: (0, 0)),
            pl.BlockSpec((n, d), lambda: (0, 0)),
        ],
        out_specs=pl.BlockSpec((1, 1), lambda: (0, 0)),
        out_shape=jax.ShapeDtypeStruct((1, 1), jnp.float32),
    )(y[:, None], y[None, :], x_soft, xb)
    return out[0, 0]


# speedup vs baseline: 9.2894x; 1.0245x over previous
import jax
import jax.numpy as jnp
from jax.experimental import pallas as pl

_LAMDA = 0.003
_K = 20
_N = 1024
_CLS = 128


def _loss_kernel(y_col_ref, y_row_ref, xs_ref, x_ref, out_ref):
    xf = x_ref[...]
    xb = xf.astype(jnp.bfloat16)
    y_col = y_col_ref[...]
    y_row = y_row_ref[...]

    cls_iota = jax.lax.broadcasted_iota(jnp.int32, (_CLS, _N), 0)
    oh = cls_iota == y_row
    ohf = oh.astype(jnp.float32)
    cnt_c = jnp.sum(ohf, axis=1, keepdims=True)

    s_cls = jnp.dot(oh.astype(jnp.bfloat16), xb,
                    preferred_element_type=jnp.float32)

    sq = jnp.sum(xf * xf, axis=1, keepdims=True)
    ssq_c = jnp.dot(ohf, sq, preferred_element_type=jnp.float32)

    term1 = jnp.sum(cnt_c * ssq_c, keepdims=True)
    term2 = jnp.sum(s_cls * s_cls, keepdims=True)
    lp_base = 2.0 * (term1 - term2)

    def _heavy():
        col = jax.lax.broadcasted_iota(jnp.int32, (_N, _N), 1)
        row = jax.lax.broadcasted_iota(jnp.int32, (_N, _N), 0)
        same = y_col == y_row
        cnt_i = jnp.sum(same.astype(jnp.float32), axis=1, keepdims=True)
        excess0 = jnp.maximum(cnt_i - 1.0 - _K, 0.0)

        g = jax.lax.dot_general(xb, xb, (((1,), (1,)), ((), ())),
                                preferred_element_type=jnp.float32)
        diag_row = jnp.sum(jnp.where(col == row, g, 0.0), axis=0,
                           keepdims=True)
        d = sq + diag_row - 2.0 * g
        valid = same & (col != row)
        dmn0 = jnp.where(valid, d, -jnp.inf)

        def cond(carry):
            return jnp.max(carry[1]) > 0.0

        def body(carry):
            dmn, ex, corr = carry
            m = jnp.max(dmn, axis=1, keepdims=True)
            corr = corr + jnp.sum(jnp.where(ex > 0.0, m, 0.0), keepdims=True)
            first = jnp.min(jnp.where(dmn == m, col, _N), axis=1,
                            keepdims=True)
            dmn = jnp.where((col == first) & (ex > 0.0), -jnp.inf, dmn)
            return dmn, jnp.maximum(ex - 1.0, 0.0), corr

        _, _, corr = jax.lax.while_loop(
            cond, body, (dmn0, excess0, jnp.zeros((1, 1), jnp.float32)))
        return corr

    corr = jax.lax.cond(jnp.max(cnt_c) > _K + 1.0, _heavy,
                        lambda: jnp.zeros((1, 1), jnp.float32))

    xs = xs_ref[...]
    mx = jnp.max(xs, axis=1, keepdims=True)
    lse = mx + jnp.log(jnp.sum(jnp.exp(xs - mx), axis=1, keepdims=True))
    lane = jax.lax.broadcasted_iota(jnp.int32, xs.shape, 1)
    lab = jnp.sum(jnp.where(lane == y_col, xs, 0.0), axis=1, keepdims=True)
    ce = jnp.sum(lse - lab, keepdims=True)

    out_ref[...] = (_LAMDA / 2.0) * (lp_base - corr) + ce / _N


def kernel(x_soft, x_feat, y):
    n, d = x_feat.shape
    y = y.astype(jnp.int32)

    out = pl.pallas_call(
        _loss_kernel,
        in_specs=[
            pl.BlockSpec((_N, 1), lambda: (0, 0)),
            pl.BlockSpec((1, _N), lambda: (0, 0)),
            pl.BlockSpec(x_soft.shape, lambda: (0, 0)),
            pl.BlockSpec((n, d), lambda: (0, 0)),
        ],
        out_specs=pl.BlockSpec((1, 1), lambda: (0, 0)),
        out_shape=jax.ShapeDtypeStruct((1, 1), jnp.float32),
    )(y[:, None], y[None, :], x_soft, x_feat)
    return out[0, 0]
